# baseline (speedup 1.0000x reference)
import functools

import jax
import jax.numpy as jnp
from jax import lax
from jax.experimental import pallas as pl
from jax.experimental.pallas import tpu as pltpu
from jax.experimental.pallas import tpu_sc as plsc

N_NODES = 10000
N_EDGES = 320000
D_EDGE = 16
D_NODE = 128
N_GRAPHS = 8

CHUNK = 128
SUB = 5
WIN = CHUNK * SUB
N_WIN = N_EDGES // WIN
N_WORKERS = 32
ITERS = (N_WIN + N_WORKERS - 1) // N_WORKERS
ROWS_PER_TILE = N_NODES // 16


def _sc_body(edgeT_hbm, eidx3_hbm, out_hbm, tbuf_v, rows_v, idx_v, zero_v, acc_sh,
             lsem0, lsem1, asem0, asem1):
    c = lax.axis_index("c")
    s = lax.axis_index("s")
    wid = s * 2 + c

    def zero_body(i, carry):
        zero_v[i, :] = jnp.zeros((D_EDGE,), jnp.float32)
        return carry

    lax.fori_loop(0, ROWS_PER_TILE, zero_body, 0)
    row0 = s * ROWS_PER_TILE
    pltpu.sync_copy(zero_v, acc_sh.at[pl.ds(row0, ROWS_PER_TILE), :])
    plsc.subcore_barrier()

    fidx = lax.iota(jnp.int32, 16)
    lsems = (lsem0, lsem1)
    asems = (asem0, asem1)

    def win_of(i):
        return wid + N_WORKERS * i

    def valid(i):
        return win_of(i) < N_WIN

    def load_descs(i, b):
        w = win_of(i)
        return (
            pltpu.make_async_copy(
                edgeT_hbm.at[:, pl.ds(w * WIN, WIN)], tbuf_v.at[b], lsems[b]),
        ) + tuple(
            pltpu.make_async_copy(
                eidx3_hbm.at[w * SUB + k, 1, :], idx_v.at[b, k], lsems[b])
            for k in range(SUB))

    def add_descs(i, b):
        return tuple(
            pltpu.make_async_copy(rows_v.at[b, pl.ds(k * CHUNK, CHUNK), :],
                                  acc_sh.at[idx_v.at[b, k]], asems[b])
            for k in range(SUB))

    def fire_loads(i, b):
        @pl.when(valid(i))
        def _():
            for d in load_descs(i, b):
                d.start()

    fire_loads(0, 0)

    def body(i2, carry):
        for b in range(2):
            i = i2 * 2 + b

            @pl.when(valid(i))
            def _():
                for d in load_descs(i, b):
                    d.wait()
                for f in range(D_EDGE):
                    fvec = jnp.full((16,), f, jnp.int32)
                    for g in range(WIN // 16):
                        v = tbuf_v[b, f, pl.ds(g * 16, 16)]
                        plsc.store_scatter(rows_v.at[b], [g * 16 + fidx, fvec], v)

            if True:
                @pl.when((i >= 1) & valid(i - 1))
                def _():
                    for d in add_descs(i - 1, 1 - b):
                        d.wait()

            fire_loads(i + 1, 1 - b)

            @pl.when(valid(i))
            def _():
                for d in add_descs(i, b):
                    d.start(add=True)

        return carry

    lax.fori_loop(0, (ITERS + 1) // 2, body, 0)
    last = ((ITERS + 1) // 2) * 2 - 1

    @pl.when(valid(last))
    def _():
        for d in add_descs(last, last % 2):
            d.wait()

    plsc.subcore_barrier()
    pltpu.sync_copy(acc_sh.at[pl.ds(row0, ROWS_PER_TILE), :],
                    out_hbm.at[c, pl.ds(row0, ROWS_PER_TILE), :])


@functools.cache
def _build_sc_scatter_add():
    mesh = plsc.VectorSubcoreMesh(core_axis_name="c", subcore_axis_name="s")
    return pl.kernel(
        _sc_body,
        mesh=mesh,
        compiler_params=pltpu.CompilerParams(use_tc_tiling_on_sc=False,
                                             needs_layout_passes=False),
        out_type=jax.ShapeDtypeStruct((2, N_NODES, D_EDGE), jnp.float32),
        scratch_types=[
            pltpu.VMEM((2, D_EDGE, WIN), jnp.float32),
            pltpu.VMEM((2, WIN, D_EDGE), jnp.float32),
            pltpu.VMEM((2, SUB, CHUNK), jnp.int32),
            pltpu.VMEM((ROWS_PER_TILE, D_EDGE), jnp.float32),
            pltpu.VMEM_SHARED((N_NODES, D_EDGE), jnp.float32),
            pltpu.SemaphoreType.DMA,
            pltpu.SemaphoreType.DMA,
            pltpu.SemaphoreType.DMA,
            pltpu.SemaphoreType.DMA,
        ],
    )


ROW_BLOCK = 1000
N_ROW_BLOCKS = N_NODES // ROW_BLOCK


def _mlp_a_body(x_ref, b_ref, u_ref, w1x_ref, w1u_ref, b1_ref, out_ref):
    xb = x_ref[...]
    bidx = b_ref[0, 0, :]
    oh = (bidx[:, None] == lax.broadcasted_iota(jnp.int32, (ROW_BLOCK, N_GRAPHS), 1)
          ).astype(jnp.float32)
    ub = jnp.dot(oh, u_ref[...], preferred_element_type=jnp.float32)
    out_ref[...] = (jnp.dot(xb, w1x_ref[...], preferred_element_type=jnp.float32)
                    + jnp.dot(ub, w1u_ref[...], preferred_element_type=jnp.float32)
                    + b1_ref[...])


_tc_mlp_a = pl.pallas_call(
    _mlp_a_body,
    grid=(N_ROW_BLOCKS,),
    in_specs=[
        pl.BlockSpec((ROW_BLOCK, D_NODE), lambda i: (i, 0)),
        pl.BlockSpec((1, 1, ROW_BLOCK), lambda i: (i, 0, 0)),
        pl.BlockSpec((N_GRAPHS, D_EDGE), lambda i: (0, 0)),
        pl.BlockSpec((D_NODE, 128), lambda i: (0, 0)),
        pl.BlockSpec((D_EDGE, 128), lambda i: (0, 0)),
        pl.BlockSpec((1, 128), lambda i: (0, 0)),
    ],
    out_specs=pl.BlockSpec((ROW_BLOCK, 128), lambda i: (i, 0)),
    out_shape=jax.ShapeDtypeStruct((N_NODES, 128), jnp.float32),
)


def _mlp_b_body(h1p_ref, m_ref, w1m_ref, w2_ref, b2_ref, w3_ref, b3_ref, out_ref):
    m = m_ref[0] + m_ref[1]
    h = h1p_ref[...] + jnp.dot(m, w1m_ref[...], preferred_element_type=jnp.float32)
    h = jnp.maximum(h, 0.0)
    h = jnp.dot(h, w2_ref[...], preferred_element_type=jnp.float32) + b2_ref[...]
    h = jnp.maximum(h, 0.0)
    out_ref[...] = jnp.dot(h, w3_ref[...], preferred_element_type=jnp.float32) + b3_ref[...]


_tc_mlp_b = pl.pallas_call(
    _mlp_b_body,
    grid=(N_ROW_BLOCKS,),
    in_specs=[
        pl.BlockSpec((ROW_BLOCK, 128), lambda i: (i, 0)),
        pl.BlockSpec((2, ROW_BLOCK, D_EDGE), lambda i: (0, i, 0)),
        pl.BlockSpec((D_EDGE, 128), lambda i: (0, 0)),
        pl.BlockSpec((128, 128), lambda i: (0, 0)),
        pl.BlockSpec((1, 128), lambda i: (0, 0)),
        pl.BlockSpec((128, 128), lambda i: (0, 0)),
        pl.BlockSpec((1, 128), lambda i: (0, 0)),
    ],
    out_specs=pl.BlockSpec((ROW_BLOCK, 128), lambda i: (i, 0)),
    out_shape=jax.ShapeDtypeStruct((N_NODES, 128), jnp.float32),
)


def kernel(x, edge_index, edge_attr, u, batch, W1, b1, W2, b2, W3, b3):
    eidx3 = edge_index.reshape(2, N_EDGES // CHUNK, CHUNK).transpose(1, 0, 2)
    parts = _build_sc_scatter_add()(edge_attr.T, eidx3)
    batch3d = batch.reshape(N_ROW_BLOCKS, 1, ROW_BLOCK)
    h1p = _tc_mlp_a(x, batch3d, u, W1[:D_NODE], W1[D_NODE + D_EDGE:],
                    b1.reshape(1, 128))
    return _tc_mlp_b(h1p, parts, W1[D_NODE:D_NODE + D_EDGE],
                     W2, b2.reshape(1, 128), W3, b3.reshape(1, 128))


# speedup vs baseline: 1.0004x; 1.0004x over previous
import functools

import jax
import jax.numpy as jnp
from jax import lax
from jax.experimental import pallas as pl
from jax.experimental.pallas import tpu as pltpu
from jax.experimental.pallas import tpu_sc as plsc

N_NODES = 10000
N_EDGES = 320000
D_EDGE = 16
D_NODE = 128
N_GRAPHS = 8

CHUNK = 128
SUB = 5
WIN = CHUNK * SUB
N_WIN = N_EDGES // WIN
N_WORKERS = 32
ITERS = (N_WIN + N_WORKERS - 1) // N_WORKERS
ROWS_PER_TILE = N_NODES // 16


def _sc_body(edgeT_hbm, eidx3_hbm, out_hbm, tbuf_v, rows_v, idx_v, zero_v, acc_sh,
             lsem0, lsem1, asem0, asem1):
    c = lax.axis_index("c")
    s = lax.axis_index("s")
    wid = s * 2 + c

    def zero_body(i, carry):
        zero_v[i, :] = jnp.zeros((D_EDGE,), jnp.float32)
        return carry

    lax.fori_loop(0, ROWS_PER_TILE, zero_body, 0)
    row0 = s * ROWS_PER_TILE
    pltpu.sync_copy(zero_v, acc_sh.at[pl.ds(row0, ROWS_PER_TILE), :])
    plsc.subcore_barrier()

    fidx = lax.iota(jnp.int32, 16)
    lsems = (lsem0, lsem1)
    asems = (asem0, asem1)

    def win_of(i):
        return wid + N_WORKERS * i

    def valid(i):
        return win_of(i) < N_WIN

    def load_descs(i, b):
        w = win_of(i)
        return (
            pltpu.make_async_copy(
                edgeT_hbm.at[:, pl.ds(w * WIN, WIN)], tbuf_v.at[b], lsems[b]),
        ) + tuple(
            pltpu.make_async_copy(
                eidx3_hbm.at[w * SUB + k, 1, :], idx_v.at[b, k], lsems[b])
            for k in range(SUB))

    def add_descs(i, b):
        return tuple(
            pltpu.make_async_copy(rows_v.at[b, pl.ds(k * CHUNK, CHUNK), :],
                                  acc_sh.at[idx_v.at[b, k]], asems[b])
            for k in range(SUB))

    def fire_loads(i, b):
        @pl.when(valid(i))
        def _():
            for d in load_descs(i, b):
                d.start()

    fire_loads(0, 0)

    def body(i2, carry):
        for b in range(2):
            i = i2 * 2 + b

            @pl.when(valid(i))
            def _():
                for d in load_descs(i, b):
                    d.wait()
                for f in range(D_EDGE):
                    fvec = jnp.full((16,), f, jnp.int32)
                    for g in range(WIN // 16):
                        v = tbuf_v[b, f, pl.ds(g * 16, 16)]
                        plsc.store_scatter(rows_v.at[b], [g * 16 + fidx, fvec], v)

            @pl.when((i >= 1) & valid(i - 1))
            def _():
                for d in add_descs(i - 1, 1 - b):
                    d.wait()

            fire_loads(i + 1, 1 - b)

            @pl.when(valid(i))
            def _():
                for d in add_descs(i, b):
                    d.start(add=True)

        return carry

    lax.fori_loop(0, (ITERS + 1) // 2, body, 0)
    last = ((ITERS + 1) // 2) * 2 - 1

    @pl.when(valid(last))
    def _():
        for d in add_descs(last, last % 2):
            d.wait()

    plsc.subcore_barrier()
    pltpu.sync_copy(acc_sh.at[pl.ds(row0, ROWS_PER_TILE), :],
                    out_hbm.at[c, pl.ds(row0, ROWS_PER_TILE), :])


@functools.cache
def _build_sc_scatter_add():
    mesh = plsc.VectorSubcoreMesh(core_axis_name="c", subcore_axis_name="s")
    return pl.kernel(
        _sc_body,
        mesh=mesh,
        compiler_params=pltpu.CompilerParams(use_tc_tiling_on_sc=False,
                                             needs_layout_passes=False),
        out_type=jax.ShapeDtypeStruct((2, N_NODES, D_EDGE), jnp.float32),
        scratch_types=[
            pltpu.VMEM((2, D_EDGE, WIN), jnp.float32),
            pltpu.VMEM((2, WIN, D_EDGE), jnp.float32),
            pltpu.VMEM((2, SUB, CHUNK), jnp.int32),
            pltpu.VMEM((ROWS_PER_TILE, D_EDGE), jnp.float32),
            pltpu.VMEM_SHARED((N_NODES, D_EDGE), jnp.float32),
            pltpu.SemaphoreType.DMA,
            pltpu.SemaphoreType.DMA,
            pltpu.SemaphoreType.DMA,
            pltpu.SemaphoreType.DMA,
        ],
    )


ROW_BLOCK = 1000
N_ROW_BLOCKS = N_NODES // ROW_BLOCK


def _mlp_a_body(x_ref, b_ref, u_ref, w1x_ref, w1u_ref, b1_ref, out_ref):
    xb = x_ref[...]
    bidx = b_ref[0, 0, :]
    oh = (bidx[:, None] == lax.broadcasted_iota(jnp.int32, (ROW_BLOCK, N_GRAPHS), 1)
          ).astype(jnp.float32)
    ub = jnp.dot(oh, u_ref[...], preferred_element_type=jnp.float32)
    out_ref[...] = (jnp.dot(xb, w1x_ref[...], preferred_element_type=jnp.float32)
                    + jnp.dot(ub, w1u_ref[...], preferred_element_type=jnp.float32)
                    + b1_ref[...])


_tc_mlp_a = pl.pallas_call(
    _mlp_a_body,
    grid=(N_ROW_BLOCKS,),
    in_specs=[
        pl.BlockSpec((ROW_BLOCK, D_NODE), lambda i: (i, 0)),
        pl.BlockSpec((1, 1, ROW_BLOCK), lambda i: (i, 0, 0)),
        pl.BlockSpec((N_GRAPHS, D_EDGE), lambda i: (0, 0)),
        pl.BlockSpec((D_NODE, 128), lambda i: (0, 0)),
        pl.BlockSpec((D_EDGE, 128), lambda i: (0, 0)),
        pl.BlockSpec((1, 128), lambda i: (0, 0)),
    ],
    out_specs=pl.BlockSpec((ROW_BLOCK, 128), lambda i: (i, 0)),
    out_shape=jax.ShapeDtypeStruct((N_NODES, 128), jnp.float32),
)


def _mlp_b_body(h1p_ref, m_ref, w1m_ref, w2_ref, b2_ref, w3_ref, b3_ref, out_ref):
    m = m_ref[0] + m_ref[1]
    h = h1p_ref[...] + jnp.dot(m, w1m_ref[...], preferred_element_type=jnp.float32)
    h = jnp.maximum(h, 0.0)
    h = jnp.dot(h, w2_ref[...], preferred_element_type=jnp.float32) + b2_ref[...]
    h = jnp.maximum(h, 0.0)
    out_ref[...] = jnp.dot(h, w3_ref[...], preferred_element_type=jnp.float32) + b3_ref[...]


_tc_mlp_b = pl.pallas_call(
    _mlp_b_body,
    grid=(N_ROW_BLOCKS,),
    in_specs=[
        pl.BlockSpec((ROW_BLOCK, 128), lambda i: (i, 0)),
        pl.BlockSpec((2, ROW_BLOCK, D_EDGE), lambda i: (0, i, 0)),
        pl.BlockSpec((D_EDGE, 128), lambda i: (0, 0)),
        pl.BlockSpec((128, 128), lambda i: (0, 0)),
        pl.BlockSpec((1, 128), lambda i: (0, 0)),
        pl.BlockSpec((128, 128), lambda i: (0, 0)),
        pl.BlockSpec((1, 128), lambda i: (0, 0)),
    ],
    out_specs=pl.BlockSpec((ROW_BLOCK, 128), lambda i: (i, 0)),
    out_shape=jax.ShapeDtypeStruct((N_NODES, 128), jnp.float32),
)


def kernel(x, edge_index, edge_attr, u, batch, W1, b1, W2, b2, W3, b3):
    eidx3 = edge_index.reshape(2, N_EDGES // CHUNK, CHUNK).transpose(1, 0, 2)
    parts = _build_sc_scatter_add()(edge_attr.T, eidx3)
    batch3d = batch.reshape(N_ROW_BLOCKS, 1, ROW_BLOCK)
    h1p = _tc_mlp_a(x, batch3d, u, W1[:D_NODE], W1[D_NODE + D_EDGE:],
                    b1.reshape(1, 128))
    return _tc_mlp_b(h1p, parts, W1[D_NODE:D_NODE + D_EDGE],
                     W2, b2.reshape(1, 128), W3, b3.reshape(1, 128))
